# baseline (speedup 1.0000x reference)
import functools

import jax
import jax.numpy as jnp
from jax import lax
from jax.experimental import pallas as pl
from jax.experimental.pallas import tpu as pltpu
from jax.experimental.pallas import tpu_sc as plsc

N = 10000
E = 320000
D = 128

NC = 2
NS = 16
NW = NC * NS
EW = E // NW
B = 100
CH = EW // B
NBUF = 2
IBUF = 4
CW = 16

ZT = 624
TAIL = N - NS * ZT
ZCHUNKS = ((0, 96), (96, 96), (192, 96), (288, 96), (384, 96), (480, 96),
           (576, 48))

BR = 1000
GRID = N // BR

_MESH = dict(core_axis_name="c", subcore_axis_name="s",
             num_cores=NC, num_subcores=NS)


def _sc_agg_body(feat, ei, out, iring, rows, acc, isems, gsems, ssems):
    cid = lax.axis_index("c")
    tid = lax.axis_index("s")
    wid = cid * NS + tid

    zvec = jnp.zeros((16,), jnp.float32)

    def _zero_row(r, _):
        for k in range(D // 16):
            rows[0, r, pl.ds(k * 16, 16)] = zvec
        return 0

    lax.fori_loop(0, B, _zero_row, 0)
    for off, sz in ZCHUNKS:
        pltpu.sync_copy(rows.at[0].at[pl.ds(0, sz)],
                        acc.at[pl.ds(tid * ZT + off, sz)])

    @pl.when(tid == NS - 1)
    def _():
        pltpu.sync_copy(rows.at[0].at[pl.ds(0, TAIL)],
                        acc.at[pl.ds(NS * ZT, TAIL)])

    plsc.subcore_barrier()

    def _idx_cp(c, q):
        return pltpu.make_async_copy(ei.at[wid * CH + c],
                                     iring.at[pl.ds(2 * q, 2)], isems[q])

    def _gat_cp(b, q):
        return pltpu.make_async_copy(feat.at[iring.at[2 * q]], rows.at[b],
                                     gsems[b])

    def _sca_cp(b, q):
        return pltpu.make_async_copy(rows.at[b], acc.at[iring.at[2 * q + 1]],
                                     ssems[b])

    for q in range(IBUF - 1):
        _idx_cp(q, q).start()
    _idx_cp(0, 0).wait()
    _gat_cp(0, 0).start()

    def _chunk(o, _):
        for b4 in range(IBUF):
            c = o * IBUF + b4
            b = b4 % NBUF
            nb = (b4 + 1) % NBUF
            nq = (b4 + 1) % IBUF
            fq = (b4 + 3) % IBUF

            _gat_cp(b, b4).wait()
            pltpu.async_copy(rows.at[b], acc.at[iring.at[2 * b4 + 1]],
                             ssems[b], add=True)

            @pl.when(c + 1 < CH)
            def _():
                @pl.when(c >= 1)
                def _():
                    _sca_cp(nb, fq).wait()

                @pl.when(c + 3 < CH)
                def _():
                    _idx_cp(c + 3, fq).start()

                _idx_cp(c + 1, nq).wait()
                _gat_cp(nb, nq).start()

        return 0

    lax.fori_loop(0, CH // IBUF, _chunk, 0)
    _sca_cp(0, 2).wait()
    _sca_cp(1, 3).wait()
    plsc.subcore_barrier()

    pltpu.sync_copy(acc.at[pl.ds(tid * ZT, ZT)],
                    out.at[cid, pl.ds(tid * ZT, ZT)])

    @pl.when(tid == NS - 1)
    def _():
        pltpu.sync_copy(acc.at[pl.ds(NS * ZT, TAIL)],
                        out.at[cid, pl.ds(NS * ZT, TAIL)])


_sc_agg = pl.kernel(
    _sc_agg_body,
    out_type=jax.ShapeDtypeStruct((NC, N, D), jnp.float32),
    mesh=plsc.VectorSubcoreMesh(**_MESH),
    scratch_types=[
        pltpu.VMEM((2 * IBUF, B), jnp.int32),
        pltpu.VMEM((NBUF, B, D), jnp.float32),
        pltpu.VMEM_SHARED((N, D), jnp.float32),
        [pltpu.SemaphoreType.DMA] * IBUF,
        [pltpu.SemaphoreType.DMA] * NBUF,
        [pltpu.SemaphoreType.DMA] * NBUF,
    ],
)


def _sc_cnt_body(ei, outc, iring, ones_v, accc, isems, csems):
    cid = lax.axis_index("c")
    tid = lax.axis_index("s")
    wid = cid * NS + tid

    zvec = jnp.zeros((16,), jnp.float32)

    def _fill(val):
        def body(r, _):
            ones_v[r, pl.ds(0, CW)] = val
            return 0
        lax.fori_loop(0, B, body, 0)

    _fill(zvec)
    for off, sz in ZCHUNKS:
        pltpu.sync_copy(ones_v.at[pl.ds(0, sz)],
                        accc.at[pl.ds(tid * ZT + off, sz)])

    @pl.when(tid == NS - 1)
    def _():
        pltpu.sync_copy(ones_v.at[pl.ds(0, TAIL)],
                        accc.at[pl.ds(NS * ZT, TAIL)])

    _fill(zvec + 1.0)
    plsc.subcore_barrier()

    def _idx_cp(c, q):
        return pltpu.make_async_copy(ei.at[wid * CH + c],
                                     iring.at[pl.ds(2 * q, 2)], isems[q])

    def _sca_cp(b, q):
        return pltpu.make_async_copy(ones_v, accc.at[iring.at[2 * q + 1]],
                                     csems[b])

    for q in range(IBUF - 1):
        _idx_cp(q, q).start()

    def _chunk(o, _):
        for b4 in range(IBUF):
            c = o * IBUF + b4
            b = b4 % 2
            nb = (b4 + 1) % 2
            fq = (b4 + 3) % IBUF

            _idx_cp(c, b4).wait()
            pltpu.async_copy(ones_v, accc.at[iring.at[2 * b4 + 1]],
                             csems[b], add=True)

            @pl.when(c + 1 < CH)
            def _():
                @pl.when(c >= 1)
                def _():
                    _sca_cp(nb, fq).wait()

                @pl.when(c + 3 < CH)
                def _():
                    _idx_cp(c + 3, fq).start()

        return 0

    lax.fori_loop(0, CH // IBUF, _chunk, 0)
    _sca_cp(0, 2).wait()
    _sca_cp(1, 3).wait()
    plsc.subcore_barrier()

    pltpu.sync_copy(accc.at[pl.ds(tid * ZT, ZT)],
                    outc.at[cid, pl.ds(tid * ZT, ZT)])

    @pl.when(tid == NS - 1)
    def _():
        pltpu.sync_copy(accc.at[pl.ds(NS * ZT, TAIL)],
                        outc.at[cid, pl.ds(NS * ZT, TAIL)])


_sc_cnt = pl.kernel(
    _sc_cnt_body,
    out_type=jax.ShapeDtypeStruct((NC, N, CW), jnp.float32),
    mesh=plsc.VectorSubcoreMesh(**_MESH),
    scratch_types=[
        pltpu.VMEM((2 * IBUF, B), jnp.int32),
        pltpu.VMEM((B, CW), jnp.float32),
        pltpu.VMEM_SHARED((N, CW), jnp.float32),
        [pltpu.SemaphoreType.DMA] * IBUF,
        [pltpu.SemaphoreType.DMA] * 2,
    ],
    compiler_params=pltpu.CompilerParams(use_tc_tiling_on_sc=False),
)


def _tc1_body(x_ref, wlT_ref, wrT_ref, bl_ref, xl_ref, xrb_ref):
    xb = x_ref[...]
    xl_ref[...] = jnp.dot(xb, wlT_ref[...], preferred_element_type=jnp.float32)
    xrb_ref[...] = (jnp.dot(xb, wrT_ref[...],
                            preferred_element_type=jnp.float32) + bl_ref[...])


def _tc2_body(p_ref, pc_ref, xrb_ref, wlT_ref, wrT_ref, bl_ref,
              hl_ref, hrb_ref, rcb_ref):
    agg = p_ref[0] + p_ref[1]
    cnt = pc_ref[0, :, 0:1] + pc_ref[1, :, 0:1]
    rc = 1.0 / jnp.maximum(cnt, 1.0)
    h = jnp.maximum(agg * rc + xrb_ref[...], 0.0)
    hl_ref[...] = jnp.dot(h, wlT_ref[...], preferred_element_type=jnp.float32)
    hrb_ref[...] = (jnp.dot(h, wrT_ref[...],
                            preferred_element_type=jnp.float32) + bl_ref[...])
    rcb_ref[...] = jnp.broadcast_to(rc, (BR, D))


def _tc3_body(p_ref, hrb_ref, rcb_ref, o_ref):
    o_ref[...] = (p_ref[0] + p_ref[1]) * rcb_ref[...] + hrb_ref[...]


_W_SPEC = pl.BlockSpec((D, D), lambda i: (0, 0))
_B_SPEC = pl.BlockSpec((1, D), lambda i: (0, 0))
_X_SPEC = pl.BlockSpec((BR, D), lambda i: (i, 0))
_P_SPEC = pl.BlockSpec((NC, BR, D), lambda i: (0, i, 0))
_PC_SPEC = pl.BlockSpec((NC, BR, CW), lambda i: (0, i, 0))

_tc1 = pl.pallas_call(
    _tc1_body,
    grid=(GRID,),
    in_specs=[_X_SPEC, _W_SPEC, _W_SPEC, _B_SPEC],
    out_specs=[_X_SPEC, _X_SPEC],
    out_shape=[jax.ShapeDtypeStruct((N, D), jnp.float32)] * 2,
)

_tc2 = pl.pallas_call(
    _tc2_body,
    grid=(GRID,),
    in_specs=[_P_SPEC, _PC_SPEC, _X_SPEC, _W_SPEC, _W_SPEC, _B_SPEC],
    out_specs=[_X_SPEC, _X_SPEC, _X_SPEC],
    out_shape=[jax.ShapeDtypeStruct((N, D), jnp.float32)] * 3,
)

_tc3 = pl.pallas_call(
    _tc3_body,
    grid=(GRID,),
    in_specs=[_P_SPEC, _X_SPEC, _X_SPEC],
    out_specs=_X_SPEC,
    out_shape=jax.ShapeDtypeStruct((N, D), jnp.float32),
)


def kernel(x, edge_index, Wl1, bl1, Wr1, Wl2, bl2, Wr2):
    ei = jnp.stack([edge_index[0].reshape(NW * CH, B),
                    edge_index[1].reshape(NW * CH, B)], axis=1)
    xl1, xrb1 = _tc1(x, Wl1.T, Wr1.T, bl1.reshape(1, D))
    p1 = _sc_agg(xl1, ei)
    pc = _sc_cnt(ei)
    hl2, hrb2, rcb = _tc2(p1, pc, xrb1, Wl2.T, Wr2.T, bl2.reshape(1, D))
    p2 = _sc_agg(hl2, ei)
    return _tc3(p2, hrb2, rcb)


# speedup vs baseline: 1.1269x; 1.1269x over previous
import functools

import jax
import jax.numpy as jnp
from jax import lax
from jax.experimental import pallas as pl
from jax.experimental.pallas import tpu as pltpu
from jax.experimental.pallas import tpu_sc as plsc

N = 10000
E = 320000
D = 128

NC = 2
NS = 16
NW = NC * NS
EW = E // NW
B = 125
CH = EW // B
NBUF = 2
IBUF = 4
CW = 16

ZT = 624
TAIL = N - NS * ZT
ZCHUNKS = ((0, 96), (96, 96), (192, 96), (288, 96), (384, 96), (480, 96),
           (576, 48))

BR = 1000
GRID = N // BR

_MESH = dict(core_axis_name="c", subcore_axis_name="s",
             num_cores=NC, num_subcores=NS)


def _sc_agg_body(feat, ei, out, iring, rows, acc, isems, gsems, ssems):
    cid = lax.axis_index("c")
    tid = lax.axis_index("s")
    wid = cid * NS + tid

    zvec = jnp.zeros((16,), jnp.float32)

    def _zero_row(r, _):
        for k in range(D // 16):
            rows[0, r, pl.ds(k * 16, 16)] = zvec
        return 0

    lax.fori_loop(0, B, _zero_row, 0)
    for off, sz in ZCHUNKS:
        pltpu.sync_copy(rows.at[0].at[pl.ds(0, sz)],
                        acc.at[pl.ds(tid * ZT + off, sz)])

    @pl.when(tid == NS - 1)
    def _():
        pltpu.sync_copy(rows.at[0].at[pl.ds(0, TAIL)],
                        acc.at[pl.ds(NS * ZT, TAIL)])

    plsc.subcore_barrier()

    def _idx_cp(c, q):
        return pltpu.make_async_copy(ei.at[wid * CH + c],
                                     iring.at[pl.ds(2 * q, 2)], isems[q])

    def _gat_cp(b, q):
        return pltpu.make_async_copy(feat.at[iring.at[2 * q]], rows.at[b],
                                     gsems[b])

    def _sca_cp(b, q):
        return pltpu.make_async_copy(rows.at[b], acc.at[iring.at[2 * q + 1]],
                                     ssems[b])

    for q in range(IBUF - 1):
        _idx_cp(q, q).start()
    _idx_cp(0, 0).wait()
    _gat_cp(0, 0).start()

    def _chunk(o, _):
        for b4 in range(IBUF):
            c = o * IBUF + b4
            b = b4 % NBUF
            nb = (b4 + 1) % NBUF
            nq = (b4 + 1) % IBUF
            fq = (b4 + 3) % IBUF

            _gat_cp(b, b4).wait()
            pltpu.async_copy(rows.at[b], acc.at[iring.at[2 * b4 + 1]],
                             ssems[b], add=True)

            @pl.when(c + 1 < CH)
            def _():
                @pl.when(c >= 1)
                def _():
                    _sca_cp(nb, fq).wait()

                @pl.when(c + 3 < CH)
                def _():
                    _idx_cp(c + 3, fq).start()

                _idx_cp(c + 1, nq).wait()
                _gat_cp(nb, nq).start()

        return 0

    lax.fori_loop(0, CH // IBUF, _chunk, 0)
    _sca_cp(0, 2).wait()
    _sca_cp(1, 3).wait()
    plsc.subcore_barrier()

    pltpu.sync_copy(acc.at[pl.ds(tid * ZT, ZT)],
                    out.at[cid, pl.ds(tid * ZT, ZT)])

    @pl.when(tid == NS - 1)
    def _():
        pltpu.sync_copy(acc.at[pl.ds(NS * ZT, TAIL)],
                        out.at[cid, pl.ds(NS * ZT, TAIL)])


_sc_agg = pl.kernel(
    _sc_agg_body,
    out_type=jax.ShapeDtypeStruct((NC, N, D), jnp.float32),
    mesh=plsc.VectorSubcoreMesh(**_MESH),
    scratch_types=[
        pltpu.VMEM((2 * IBUF, B), jnp.int32),
        pltpu.VMEM((NBUF, B, D), jnp.float32),
        pltpu.VMEM_SHARED((N, D), jnp.float32),
        [pltpu.SemaphoreType.DMA] * IBUF,
        [pltpu.SemaphoreType.DMA] * NBUF,
        [pltpu.SemaphoreType.DMA] * NBUF,
    ],
)


def _sc_cnt_body(ei, outc, iring, ones_v, accc, isems, csems):
    cid = lax.axis_index("c")
    tid = lax.axis_index("s")
    wid = cid * NS + tid

    zvec = jnp.zeros((16,), jnp.float32)

    def _fill(val):
        def body(r, _):
            ones_v[r, pl.ds(0, CW)] = val
            return 0
        lax.fori_loop(0, B, body, 0)

    _fill(zvec)
    for off, sz in ZCHUNKS:
        pltpu.sync_copy(ones_v.at[pl.ds(0, sz)],
                        accc.at[pl.ds(tid * ZT + off, sz)])

    @pl.when(tid == NS - 1)
    def _():
        pltpu.sync_copy(ones_v.at[pl.ds(0, TAIL)],
                        accc.at[pl.ds(NS * ZT, TAIL)])

    _fill(zvec + 1.0)
    plsc.subcore_barrier()

    def _idx_cp(c, q):
        return pltpu.make_async_copy(ei.at[wid * CH + c],
                                     iring.at[pl.ds(2 * q, 2)], isems[q])

    def _sca_cp(b, q):
        return pltpu.make_async_copy(ones_v, accc.at[iring.at[2 * q + 1]],
                                     csems[b])

    for q in range(IBUF - 1):
        _idx_cp(q, q).start()

    def _chunk(o, _):
        for b4 in range(IBUF):
            c = o * IBUF + b4
            b = b4 % 2
            nb = (b4 + 1) % 2
            fq = (b4 + 3) % IBUF

            _idx_cp(c, b4).wait()
            pltpu.async_copy(ones_v, accc.at[iring.at[2 * b4 + 1]],
                             csems[b], add=True)

            @pl.when(c + 1 < CH)
            def _():
                @pl.when(c >= 1)
                def _():
                    _sca_cp(nb, fq).wait()

                @pl.when(c + 3 < CH)
                def _():
                    _idx_cp(c + 3, fq).start()

        return 0

    lax.fori_loop(0, CH // IBUF, _chunk, 0)
    _sca_cp(0, 2).wait()
    _sca_cp(1, 3).wait()
    plsc.subcore_barrier()

    pltpu.sync_copy(accc.at[pl.ds(tid * ZT, ZT)],
                    outc.at[cid, pl.ds(tid * ZT, ZT)])

    @pl.when(tid == NS - 1)
    def _():
        pltpu.sync_copy(accc.at[pl.ds(NS * ZT, TAIL)],
                        outc.at[cid, pl.ds(NS * ZT, TAIL)])


_sc_cnt = pl.kernel(
    _sc_cnt_body,
    out_type=jax.ShapeDtypeStruct((NC, N, CW), jnp.float32),
    mesh=plsc.VectorSubcoreMesh(**_MESH),
    scratch_types=[
        pltpu.VMEM((2 * IBUF, B), jnp.int32),
        pltpu.VMEM((B, CW), jnp.float32),
        pltpu.VMEM_SHARED((N, CW), jnp.float32),
        [pltpu.SemaphoreType.DMA] * IBUF,
        [pltpu.SemaphoreType.DMA] * 2,
    ],
    compiler_params=pltpu.CompilerParams(use_tc_tiling_on_sc=False),
)


def _tc1_body(x_ref, wlT_ref, wrT_ref, bl_ref, xl_ref, xrb_ref):
    xb = x_ref[...]
    xl_ref[...] = jnp.dot(xb, wlT_ref[...], preferred_element_type=jnp.float32)
    xrb_ref[...] = (jnp.dot(xb, wrT_ref[...],
                            preferred_element_type=jnp.float32) + bl_ref[...])


def _tc2_body(p_ref, pc_ref, xrb_ref, wlT_ref, wrT_ref, bl_ref,
              hl_ref, hrb_ref, rcb_ref):
    agg = p_ref[0] + p_ref[1]
    cnt = pc_ref[0, :, 0:1] + pc_ref[1, :, 0:1]
    rc = 1.0 / jnp.maximum(cnt, 1.0)
    h = jnp.maximum(agg * rc + xrb_ref[...], 0.0)
    hl_ref[...] = jnp.dot(h, wlT_ref[...], preferred_element_type=jnp.float32)
    hrb_ref[...] = (jnp.dot(h, wrT_ref[...],
                            preferred_element_type=jnp.float32) + bl_ref[...])
    rcb_ref[...] = jnp.broadcast_to(rc, (BR, D))


def _tc3_body(p_ref, hrb_ref, rcb_ref, o_ref):
    o_ref[...] = (p_ref[0] + p_ref[1]) * rcb_ref[...] + hrb_ref[...]


_W_SPEC = pl.BlockSpec((D, D), lambda i: (0, 0))
_B_SPEC = pl.BlockSpec((1, D), lambda i: (0, 0))
_X_SPEC = pl.BlockSpec((BR, D), lambda i: (i, 0))
_P_SPEC = pl.BlockSpec((NC, BR, D), lambda i: (0, i, 0))
_PC_SPEC = pl.BlockSpec((NC, BR, CW), lambda i: (0, i, 0))

_tc1 = pl.pallas_call(
    _tc1_body,
    grid=(GRID,),
    in_specs=[_X_SPEC, _W_SPEC, _W_SPEC, _B_SPEC],
    out_specs=[_X_SPEC, _X_SPEC],
    out_shape=[jax.ShapeDtypeStruct((N, D), jnp.float32)] * 2,
)

_tc2 = pl.pallas_call(
    _tc2_body,
    grid=(GRID,),
    in_specs=[_P_SPEC, _PC_SPEC, _X_SPEC, _W_SPEC, _W_SPEC, _B_SPEC],
    out_specs=[_X_SPEC, _X_SPEC, _X_SPEC],
    out_shape=[jax.ShapeDtypeStruct((N, D), jnp.float32)] * 3,
)

_tc3 = pl.pallas_call(
    _tc3_body,
    grid=(GRID,),
    in_specs=[_P_SPEC, _X_SPEC, _X_SPEC],
    out_specs=_X_SPEC,
    out_shape=jax.ShapeDtypeStruct((N, D), jnp.float32),
)


def kernel(x, edge_index, Wl1, bl1, Wr1, Wl2, bl2, Wr2):
    ei = jnp.stack([edge_index[0].reshape(NW * CH, B),
                    edge_index[1].reshape(NW * CH, B)], axis=1)
    xl1, xrb1 = _tc1(x, Wl1.T, Wr1.T, bl1.reshape(1, D))
    p1 = _sc_agg(xl1, ei)
    pc = _sc_cnt(ei)
    hl2, hrb2, rcb = _tc2(p1, pc, xrb1, Wl2.T, Wr2.T, bl2.reshape(1, D))
    p2 = _sc_agg(hl2, ei)
    return _tc3(p2, hrb2, rcb)
